# Initial kernel scaffold; baseline (speedup 1.0000x reference)
import functools

import jax
import jax.numpy as jnp
from jax import lax
from jax.experimental import pallas as pl
from jax.experimental.pallas import tpu as pltpu
from jax.experimental.pallas import tpu_sc as plsc

NC = 2
NS = 16
LN = 16
EMB = 32
FEAT = 64
MOM = 0.5
LEAK = 0.1
CHUNK = 512


def _pick_row_chunk(r):
    for ch in (448, 256, 224, 128, 112, 64, 32):
        if r % ch == 0:
            return ch
    return 32


def _spmm_momentum(src2, dst2, val2, x, npad, ep):
    n2 = npad // 2
    rpt = n2 // NS
    gt = ep // CHUNK // NS
    ch = _pick_row_chunk(rpt)
    nch = rpt // ch
    mesh = plsc.VectorSubcoreMesh(core_axis_name="c", subcore_axis_name="s",
                                  num_cores=NC, num_subcores=NS)

    @functools.partial(
        pl.kernel,
        out_type=jax.ShapeDtypeStruct((npad, EMB), jnp.float32),
        mesh=mesh,
        scratch_types=[
            pltpu.VMEM_SHARED((n2 + 16, EMB), jnp.float32),
            pltpu.VMEM((4, 128), jnp.int32),
            pltpu.VMEM((4, 128), jnp.int32),
            pltpu.VMEM((4, 128), jnp.float32),
            pltpu.VMEM((4, 128), jnp.int32),
            pltpu.VMEM((CHUNK, EMB), jnp.float32),
            pltpu.VMEM((None, EMB), jnp.float32) if False else None,
            pltpu.SemaphoreType.DMA,
        ],
    )
    def k(x_hbm, src_hbm, dst_hbm, val_hbm, out_hbm,
          acc, srcb, dstb, valb, lidxb, rowsb, bufA, sem):
        pass

    return None


# speedup vs baseline: 4.5708x; 4.5708x over previous
import functools

import jax
import jax.numpy as jnp
from jax import lax
from jax.experimental import pallas as pl
from jax.experimental.pallas import tpu as pltpu
from jax.experimental.pallas import tpu_sc as plsc

NC = 2
NS = 16
LN = 16
EMB = 32
FEAT = 64
MOM = 0.5
LEAK = 0.1
CHUNK = 256
JR = CHUNK // 128


def _pick_row_chunk(r):
    for c in (128, 112, 64, 32):
        if r % c == 0:
            return c
    return 32


def _spmm_momentum(src2, dst2, val2, x, npad, ep):
    n2 = npad // 2
    rpt = n2 // NS
    gt = ep // CHUNK // NS
    ch = _pick_row_chunk(rpt)
    nch = rpt // ch
    mesh = plsc.VectorSubcoreMesh(core_axis_name="c", subcore_axis_name="s",
                                  num_cores=NC, num_subcores=NS)

    @functools.partial(
        pl.kernel,
        out_type=jax.ShapeDtypeStruct((npad, EMB), jnp.float32),
        mesh=mesh,
        compiler_params=pltpu.CompilerParams(use_tc_tiling_on_sc=False),
        scratch_types=[
            pltpu.VMEM_SHARED((n2 + 16, EMB), jnp.float32),
            pltpu.VMEM((JR, 128), jnp.int32),
            pltpu.VMEM((JR, 128), jnp.int32),
            pltpu.VMEM((JR, 128), jnp.float32),
            pltpu.VMEM((JR, 128), jnp.int32),
            pltpu.VMEM((CHUNK, EMB), jnp.float32),
            pltpu.VMEM((ch, EMB), jnp.float32),
            pltpu.VMEM((ch, EMB), jnp.float32),
            pltpu.SemaphoreType.DMA,
        ],
    )
    def k(x_hbm, src_hbm, dst_hbm, val_hbm, out_hbm,
          acc, srcb, dstb, valb, lidxb, rowsb, bufA, bufB, sem):
        c = lax.axis_index("c")
        s = lax.axis_index("s")
        base = c * n2

        pltpu.sync_copy(x_hbm.at[pl.ds(base + s * rpt, rpt)],
                        acc.at[pl.ds(s * rpt, rpt)])
        plsc.subcore_barrier()

        def chunk_body(i, _):
            g = i * NS + s
            r0 = g * JR
            pltpu.sync_copy(src_hbm.at[pl.ds(r0, JR)], srcb)
            pltpu.sync_copy(dst_hbm.at[pl.ds(r0, JR)], dstb)
            pltpu.sync_copy(val_hbm.at[pl.ds(r0, JR)], valb)
            cps = [pltpu.async_copy(x_hbm.at[srcb.at[j]],
                                    rowsb.at[pl.ds(j * 128, 128)], sem)
                   for j in range(JR)]
            for cp in cps:
                cp.wait()

            def grp(kk, _):
                j = kk // 8
                off = (kk % 8) * LN
                dd = dstb[j, pl.ds(off, LN)]
                ll = dd - base
                inr = (ll >= 0) & (ll < n2)
                lidxb[j, pl.ds(off, LN)] = jnp.where(inr, ll, n2)
                vv = valb[j, pl.ds(off, LN)]
                e0 = kk * LN
                for lane in range(LN):
                    v = vv[lane]
                    e = e0 + lane
                    rowsb[e, pl.ds(0, LN)] = rowsb[e, pl.ds(0, LN)] * v
                    rowsb[e, pl.ds(LN, LN)] = rowsb[e, pl.ds(LN, LN)] * v
                return 0

            lax.fori_loop(0, CHUNK // LN, grp, 0)
            for j in range(JR):
                pltpu.sync_copy(rowsb.at[pl.ds(j * 128, 128)],
                                acc.at[lidxb.at[j]], add=True)
            return 0

        lax.fori_loop(0, gt, chunk_body, 0)
        plsc.subcore_barrier()

        def out_body(t, _):
            lo = s * rpt + t * ch
            pltpu.sync_copy(acc.at[pl.ds(lo, ch)], bufA)
            pltpu.sync_copy(x_hbm.at[pl.ds(base + lo, ch)], bufB)

            def vec(q, _):
                rr = q // 2
                hh = (q % 2) * LN
                a = bufA[rr, pl.ds(hh, LN)]
                b = bufB[rr, pl.ds(hh, LN)]
                bufA[rr, pl.ds(hh, LN)] = a - (1.0 - MOM) * b
                return 0

            lax.fori_loop(0, ch * 2, vec, 0)
            pltpu.sync_copy(bufA, out_hbm.at[pl.ds(base + lo, ch)])
            return 0

        lax.fori_loop(0, nch, out_body, 0)

    return k(x, src2, dst2, val2)


def _batch_lookup(sid, eid, s0, s1, s2, e0, e1, e2, ki):
    b = sid.shape[0]
    w = b // (NC * NS)
    mesh = plsc.VectorSubcoreMesh(core_axis_name="c", subcore_axis_name="s",
                                  num_cores=NC, num_subcores=NS)

    @functools.partial(
        pl.kernel,
        out_type=[jax.ShapeDtypeStruct((b, EMB), jnp.float32)] * 3,
        mesh=mesh,
        compiler_params=pltpu.CompilerParams(use_tc_tiling_on_sc=False),
        scratch_types=[
            pltpu.VMEM((w,), jnp.int32),
            pltpu.VMEM((w, EMB), jnp.float32),
            pltpu.VMEM((w, EMB), jnp.float32),
            pltpu.VMEM((w, EMB), jnp.float32),
            pltpu.SemaphoreType.DMA,
        ],
    )
    def k(sid_h, eid_h, s0_h, s1_h, s2_h, e0_h, e1_h, e2_h, ki_h,
          bstu_h, bexer_h, kib_h, idb, g0, g1, g2, sem):
        c = lax.axis_index("c")
        s = lax.axis_index("s")
        wid = s * NC + c
        base = wid * w

        def mean3():
            def vec(q, _):
                rr = q // 2
                hh = (q % 2) * LN
                a = g0[rr, pl.ds(hh, LN)]
                bb = g1[rr, pl.ds(hh, LN)]
                cc = g2[rr, pl.ds(hh, LN)]
                g0[rr, pl.ds(hh, LN)] = (a + bb + cc) * (1.0 / 3.0)
                return 0
            lax.fori_loop(0, w * 2, vec, 0)

        pltpu.sync_copy(sid_h.at[pl.ds(base, w)], idb)
        for tab, dst in ((s0_h, g0), (s1_h, g1), (s2_h, g2)):
            pltpu.async_copy(tab.at[idb], dst, sem).wait()
        mean3()
        pltpu.sync_copy(g0, bstu_h.at[pl.ds(base, w)])

        pltpu.sync_copy(eid_h.at[pl.ds(base, w)], idb)
        for tab, dst in ((e0_h, g0), (e1_h, g1), (e2_h, g2)):
            pltpu.async_copy(tab.at[idb], dst, sem).wait()
        mean3()
        pltpu.sync_copy(g0, bexer_h.at[pl.ds(base, w)])

        pltpu.async_copy(ki_h.at[idb], g1, sem).wait()
        pltpu.sync_copy(g1, kib_h.at[pl.ds(base, w)])

    return k(sid, eid, s0, s1, s2, e0, e1, e2, ki)


def _dense_tail(bstu, bexer, k0, k1, k2, wsT, bs, weT, be, wkT, bk, wd, bd):
    b = bstu.shape[0]
    kn = k0.shape[0]
    blk = 1024
    grid = b // blk

    def body(bs_ref, be_ref, k0_ref, k1_ref, k2_ref, wsT_ref, bsb_ref,
             weT_ref, beb_ref, wkT_ref, bkb_ref, wd_ref, bd_ref,
             st_ref, dt_ref, disc_ref, kt_ref):
        kf = (k0_ref[...] + k1_ref[...] + k2_ref[...]) * (1.0 / 3.0)
        kt = jnp.dot(kf, wkT_ref[...], preferred_element_type=jnp.float32,
                     precision=lax.Precision.HIGHEST) + bkb_ref[...]
        kt = jnp.where(kt > 0, kt, LEAK * kt)
        kt_ref[...] = kt

        st = jnp.dot(bs_ref[...], wsT_ref[...],
                     preferred_element_type=jnp.float32,
                     precision=lax.Precision.HIGHEST) + bsb_ref[...]
        st = jnp.where(st > 0, st, LEAK * st)
        st_ref[...] = lax.dot_general(
            st, kt, (((1,), (1,)), ((), ())),
            preferred_element_type=jnp.float32,
            precision=lax.Precision.HIGHEST)

        dt = jnp.dot(be_ref[...], weT_ref[...],
                     preferred_element_type=jnp.float32,
                     precision=lax.Precision.HIGHEST) + beb_ref[...]
        dt = jnp.where(dt > 0, dt, LEAK * dt)
        dt_ref[...] = lax.dot_general(
            dt, kt, (((1,), (1,)), ((), ())),
            preferred_element_type=jnp.float32,
            precision=lax.Precision.HIGHEST)

        dv = jnp.sum(be_ref[...] * wd_ref[...], axis=1) + bd_ref[0]
        disc_ref[...] = jax.nn.sigmoid(dv)[:, None] * jnp.ones(
            (1, 128), jnp.float32)

    full = lambda i: (0, 0)
    return pl.pallas_call(
        body,
        grid=(grid,),
        in_specs=[
            pl.BlockSpec((blk, EMB), lambda i: (i, 0)),
            pl.BlockSpec((blk, EMB), lambda i: (i, 0)),
            pl.BlockSpec((kn, EMB), full),
            pl.BlockSpec((kn, EMB), full),
            pl.BlockSpec((kn, EMB), full),
            pl.BlockSpec((EMB, FEAT), full),
            pl.BlockSpec((FEAT,), lambda i: (0,)),
            pl.BlockSpec((EMB, FEAT), full),
            pl.BlockSpec((FEAT,), lambda i: (0,)),
            pl.BlockSpec((EMB, FEAT), full),
            pl.BlockSpec((FEAT,), lambda i: (0,)),
            pl.BlockSpec((1, EMB), full),
            pl.BlockSpec((1,), lambda i: (0,)),
        ],
        out_specs=[
            pl.BlockSpec((blk, kn), lambda i: (i, 0)),
            pl.BlockSpec((blk, kn), lambda i: (i, 0)),
            pl.BlockSpec((blk, 128), lambda i: (i, 0)),
            pl.BlockSpec((kn, FEAT), full),
        ],
        out_shape=[
            jax.ShapeDtypeStruct((b, kn), jnp.float32),
            jax.ShapeDtypeStruct((b, kn), jnp.float32),
            jax.ShapeDtypeStruct((b, 128), jnp.float32),
            jax.ShapeDtypeStruct((kn, FEAT), jnp.float32),
        ],
    )(bstu, bexer, k0, k1, k2, wsT, bs, weT, be, wkT, bk, wd, bd)


def _pad_rows(x, npad):
    n = x.shape[0]
    if n == npad:
        return x
    return jnp.pad(x, ((0, npad - n), (0, 0)))


def _prep_edges(idx, val, ep):
    e = idx.shape[1]
    src = idx[1].astype(jnp.int32)
    dst = idx[0].astype(jnp.int32)
    val = val.astype(jnp.float32)
    if e != ep:
        src = jnp.pad(src, (0, ep - e))
        dst = jnp.pad(dst, (0, ep - e))
        val = jnp.pad(val, (0, ep - e))
    shape2 = (ep // 128, 128)
    return src.reshape(shape2), dst.reshape(shape2), val.reshape(shape2)


def _conv_tables(emb, idx, val, npad, ep):
    x0 = _pad_rows(emb.astype(jnp.float32), npad)
    src2, dst2, val2 = _prep_edges(idx, val, ep)
    x1 = _spmm_momentum(src2, dst2, val2, x0, npad, ep)
    x2 = _spmm_momentum(src2, dst2, val2, x1, npad, ep)
    return x0, x1, x2


def _round_up(n, m):
    return (n + m - 1) // m * m


def kernel(student_id, exercise_id, q_mask, stu_adj_idx, stu_adj_val,
           exer_adj_idx, exer_adj_val, know_adj_idx, know_adj_val,
           stu_emb, exer_emb, know_emb, ki_emb, Ws, bs, We, be, Wk, bk,
           Wd, bd):
    k_num = know_emb.shape[0]

    s_pad = _round_up(stu_emb.shape[0], 512)
    e_pad = _round_up(exer_emb.shape[0], 512)
    k_pad = _round_up(k_num, 512)
    se_pad = _round_up(stu_adj_idx.shape[1], CHUNK * NS)
    ee_pad = _round_up(exer_adj_idx.shape[1], CHUNK * NS)
    ke_pad = _round_up(know_adj_idx.shape[1], CHUNK * NS)

    s0, s1, s2 = _conv_tables(stu_emb, stu_adj_idx, stu_adj_val, s_pad, se_pad)
    e0, e1, e2 = _conv_tables(exer_emb, exer_adj_idx, exer_adj_val, e_pad, ee_pad)
    k0, k1, k2 = _conv_tables(know_emb, know_adj_idx, know_adj_val, k_pad, ke_pad)

    sid = student_id.astype(jnp.int32)
    eid = exercise_id.astype(jnp.int32)
    bstu, bexer, kib = _batch_lookup(sid, eid, s0, s1, s2, e0, e1, e2,
                                     ki_emb.astype(jnp.float32))

    st, dt, disc2d, kt = _dense_tail(
        bstu, bexer, k0[:k_num], k1[:k_num], k2[:k_num],
        Ws.T, bs, We.T, be, Wk.T, bk, Wd, bd)
    return st, dt, disc2d[:, :1], kt, kib


# speedup vs baseline: 5.5666x; 1.2179x over previous
import functools

import jax
import jax.numpy as jnp
from jax import lax
from jax.experimental import pallas as pl
from jax.experimental.pallas import tpu as pltpu
from jax.experimental.pallas import tpu_sc as plsc

NC = 2
NS = 16
LN = 16
EMB = 32
FEAT = 64
MOM = 0.5
LEAK = 0.1
CHUNK = 256
JR = CHUNK // 128


def _pick_row_chunk(r):
    for c in (128, 112, 64, 32):
        if r % c == 0:
            return c
    return 32


def _spmm_momentum(edges3, vals2, x, npad, ep, chunk):
    n2 = npad // 2
    rpt = n2 // NS
    gt = ep // chunk // NS
    jr = chunk // 128
    ch = _pick_row_chunk(rpt)
    nch = rpt // ch
    grp_n = chunk // LN
    mesh = plsc.VectorSubcoreMesh(core_axis_name="c", subcore_axis_name="s",
                                  num_cores=NC, num_subcores=NS)

    @functools.partial(
        pl.kernel,
        out_type=jax.ShapeDtypeStruct((npad, EMB), jnp.float32),
        mesh=mesh,
        compiler_params=pltpu.CompilerParams(use_tc_tiling_on_sc=False),
        scratch_types=[
            pltpu.VMEM_SHARED((n2 + 16, EMB), jnp.float32),
            pltpu.VMEM((jr, 2, 128), jnp.int32),
            pltpu.VMEM((jr, 2, 128), jnp.int32),
            pltpu.VMEM((jr, 128), jnp.float32),
            pltpu.VMEM((jr, 128), jnp.float32),
            pltpu.VMEM((jr, 128), jnp.int32),
            pltpu.VMEM((jr, 128), jnp.int32),
            pltpu.VMEM((chunk, EMB), jnp.float32),
            pltpu.VMEM((chunk, EMB), jnp.float32),
            pltpu.VMEM((ch, EMB), jnp.float32),
            pltpu.VMEM((ch, EMB), jnp.float32),
            pltpu.SemaphoreType.DMA,
            pltpu.SemaphoreType.DMA,
            pltpu.SemaphoreType.DMA,
            pltpu.SemaphoreType.DMA,
            pltpu.SemaphoreType.DMA,
            pltpu.SemaphoreType.DMA,
        ],
    )
    def k(x_hbm, edges_hbm, vals_hbm, out_hbm,
          acc, idxb0, idxb1, valb0, valb1, lidxb0, lidxb1, rowsb0, rowsb1,
          bufA, bufB, ix_sem0, ix_sem1, g_sem0, g_sem1, s_sem0, s_sem1):
        c = lax.axis_index("c")
        s = lax.axis_index("s")
        base = c * n2
        idxb = (idxb0, idxb1)
        valb = (valb0, valb1)
        lidxb = (lidxb0, lidxb1)
        rowsb = (rowsb0, rowsb1)
        ix_sem = (ix_sem0, ix_sem1)
        g_sem = (g_sem0, g_sem1)
        s_sem = (s_sem0, s_sem1)

        pltpu.sync_copy(x_hbm.at[pl.ds(base + s * rpt, rpt)],
                        acc.at[pl.ds(s * rpt, rpt)])
        plsc.subcore_barrier()

        def issue_idx(cc, b):
            pltpu.async_copy(edges_hbm.at[pl.ds(cc * jr, jr)], idxb[b],
                             ix_sem[b])
            pltpu.async_copy(vals_hbm.at[pl.ds(cc * jr, jr)], valb[b],
                             ix_sem[b])

        def wait_idx(b):
            pltpu.make_async_copy(edges_hbm.at[pl.ds(0, jr)], idxb[b],
                                  ix_sem[b]).wait()
            pltpu.make_async_copy(vals_hbm.at[pl.ds(0, jr)], valb[b],
                                  ix_sem[b]).wait()

        def issue_gather(b):
            for j in range(jr):
                pltpu.async_copy(x_hbm.at[idxb[b].at[j, 0]],
                                 rowsb[b].at[pl.ds(j * 128, 128)], g_sem[b])

        def wait_gather(b):
            for j in range(jr):
                pltpu.make_async_copy(x_hbm.at[idxb[b].at[j, 0]],
                                      rowsb[b].at[pl.ds(j * 128, 128)],
                                      g_sem[b]).wait()

        def issue_scatter(b):
            for j in range(jr):
                pltpu.async_copy(rowsb[b].at[pl.ds(j * 128, 128)],
                                 acc.at[lidxb[b].at[j]], s_sem[b], add=True)

        def wait_scatter(b):
            for j in range(jr):
                pltpu.make_async_copy(rowsb[b].at[pl.ds(j * 128, 128)],
                                      acc.at[lidxb[b].at[j]],
                                      s_sem[b]).wait()

        def scale(b):
            ib = idxb[b]
            vb = valb[b]
            lb = lidxb[b]
            rb = rowsb[b]

            def grp(kk, _):
                j = kk // 8
                off = (kk % 8) * LN
                dd = ib[j, 1, pl.ds(off, LN)]
                ll = dd - base
                inr = (ll >= 0) & (ll < n2)
                lb[j, pl.ds(off, LN)] = jnp.where(inr, ll, n2)
                vv = vb[j, pl.ds(off, LN)]
                e0 = kk * LN
                for lane in range(LN):
                    v = vv[lane]
                    e = e0 + lane
                    rb[e, pl.ds(0, LN)] = rb[e, pl.ds(0, LN)] * v
                    rb[e, pl.ds(LN, LN)] = rb[e, pl.ds(LN, LN)] * v
                return 0

            lax.fori_loop(0, grp_n, grp, 0)

        def chunk_of(i):
            return i * NS + s

        issue_idx(chunk_of(0), 0)
        issue_idx(chunk_of(1), 1)
        wait_idx(0)
        issue_gather(0)

        def body(i2, _):
            i = 2 * i2

            @pl.when(i2 >= 1)
            def _():
                wait_scatter(1)
            wait_idx(1)
            issue_gather(1)

            wait_gather(0)
            scale(0)
            issue_scatter(0)

            @pl.when(i + 2 < gt)
            def _():
                issue_idx(chunk_of(i + 2), 0)
                wait_scatter(0)
                wait_idx(0)
                issue_gather(0)

            wait_gather(1)
            scale(1)
            issue_scatter(1)

            @pl.when(i + 3 < gt)
            def _():
                issue_idx(chunk_of(i + 3), 1)
            return 0

        lax.fori_loop(0, gt // 2, body, 0)
        wait_scatter(0)
        wait_scatter(1)
        plsc.subcore_barrier()

        def out_body(t, _):
            lo = s * rpt + t * ch
            pltpu.sync_copy(acc.at[pl.ds(lo, ch)], bufA)
            pltpu.sync_copy(x_hbm.at[pl.ds(base + lo, ch)], bufB)

            def vec(q, _):
                rr = q // 2
                hh = (q % 2) * LN
                a = bufA[rr, pl.ds(hh, LN)]
                b = bufB[rr, pl.ds(hh, LN)]
                bufA[rr, pl.ds(hh, LN)] = a - (1.0 - MOM) * b
                return 0

            lax.fori_loop(0, ch * 2, vec, 0)
            pltpu.sync_copy(bufA, out_hbm.at[pl.ds(base + lo, ch)])
            return 0

        lax.fori_loop(0, nch, out_body, 0)

    return k(x, edges3, vals2)


def _batch_lookup(sid, eid, s0, s1, s2, e0, e1, e2, ki):
    b = sid.shape[0]
    w = b // (NC * NS)
    mesh = plsc.VectorSubcoreMesh(core_axis_name="c", subcore_axis_name="s",
                                  num_cores=NC, num_subcores=NS)

    @functools.partial(
        pl.kernel,
        out_type=[jax.ShapeDtypeStruct((b, EMB), jnp.float32)] * 3,
        mesh=mesh,
        compiler_params=pltpu.CompilerParams(use_tc_tiling_on_sc=False),
        scratch_types=[
            pltpu.VMEM((w,), jnp.int32),
            pltpu.VMEM((w, EMB), jnp.float32),
            pltpu.VMEM((w, EMB), jnp.float32),
            pltpu.VMEM((w, EMB), jnp.float32),
            pltpu.SemaphoreType.DMA,
        ],
    )
    def k(sid_h, eid_h, s0_h, s1_h, s2_h, e0_h, e1_h, e2_h, ki_h,
          bstu_h, bexer_h, kib_h, idb, g0, g1, g2, sem):
        c = lax.axis_index("c")
        s = lax.axis_index("s")
        wid = s * NC + c
        base = wid * w

        def mean3():
            def vec(q, _):
                rr = q // 2
                hh = (q % 2) * LN
                a = g0[rr, pl.ds(hh, LN)]
                bb = g1[rr, pl.ds(hh, LN)]
                cc = g2[rr, pl.ds(hh, LN)]
                g0[rr, pl.ds(hh, LN)] = (a + bb + cc) * (1.0 / 3.0)
                return 0
            lax.fori_loop(0, w * 2, vec, 0)

        pltpu.sync_copy(sid_h.at[pl.ds(base, w)], idb)
        for tab, dst in ((s0_h, g0), (s1_h, g1), (s2_h, g2)):
            pltpu.async_copy(tab.at[idb], dst, sem).wait()
        mean3()
        pltpu.sync_copy(g0, bstu_h.at[pl.ds(base, w)])

        pltpu.sync_copy(eid_h.at[pl.ds(base, w)], idb)
        for tab, dst in ((e0_h, g0), (e1_h, g1), (e2_h, g2)):
            pltpu.async_copy(tab.at[idb], dst, sem).wait()
        mean3()
        pltpu.sync_copy(g0, bexer_h.at[pl.ds(base, w)])

        pltpu.async_copy(ki_h.at[idb], g1, sem).wait()
        pltpu.sync_copy(g1, kib_h.at[pl.ds(base, w)])

    return k(sid, eid, s0, s1, s2, e0, e1, e2, ki)


def _dense_tail(bstu, bexer, k0, k1, k2, wsT, bs, weT, be, wkT, bk, wd, bd):
    b = bstu.shape[0]
    kn = k0.shape[0]
    blk = 1024
    grid = b // blk

    def body(bs_ref, be_ref, k0_ref, k1_ref, k2_ref, wsT_ref, bsb_ref,
             weT_ref, beb_ref, wkT_ref, bkb_ref, wd_ref, bd_ref,
             st_ref, dt_ref, disc_ref, kt_ref):
        kf = (k0_ref[...] + k1_ref[...] + k2_ref[...]) * (1.0 / 3.0)
        kt = jnp.dot(kf, wkT_ref[...], preferred_element_type=jnp.float32,
                     precision=lax.Precision.HIGHEST) + bkb_ref[...]
        kt = jnp.where(kt > 0, kt, LEAK * kt)
        kt_ref[...] = kt

        st = jnp.dot(bs_ref[...], wsT_ref[...],
                     preferred_element_type=jnp.float32,
                     precision=lax.Precision.HIGHEST) + bsb_ref[...]
        st = jnp.where(st > 0, st, LEAK * st)
        st_ref[...] = lax.dot_general(
            st, kt, (((1,), (1,)), ((), ())),
            preferred_element_type=jnp.float32,
            precision=lax.Precision.HIGHEST)

        dt = jnp.dot(be_ref[...], weT_ref[...],
                     preferred_element_type=jnp.float32,
                     precision=lax.Precision.HIGHEST) + beb_ref[...]
        dt = jnp.where(dt > 0, dt, LEAK * dt)
        dt_ref[...] = lax.dot_general(
            dt, kt, (((1,), (1,)), ((), ())),
            preferred_element_type=jnp.float32,
            precision=lax.Precision.HIGHEST)

        dv = jnp.sum(be_ref[...] * wd_ref[...], axis=1) + bd_ref[0]
        disc_ref[...] = jax.nn.sigmoid(dv)[:, None] * jnp.ones(
            (1, 128), jnp.float32)

    full = lambda i: (0, 0)
    return pl.pallas_call(
        body,
        grid=(grid,),
        in_specs=[
            pl.BlockSpec((blk, EMB), lambda i: (i, 0)),
            pl.BlockSpec((blk, EMB), lambda i: (i, 0)),
            pl.BlockSpec((kn, EMB), full),
            pl.BlockSpec((kn, EMB), full),
            pl.BlockSpec((kn, EMB), full),
            pl.BlockSpec((EMB, FEAT), full),
            pl.BlockSpec((FEAT,), lambda i: (0,)),
            pl.BlockSpec((EMB, FEAT), full),
            pl.BlockSpec((FEAT,), lambda i: (0,)),
            pl.BlockSpec((EMB, FEAT), full),
            pl.BlockSpec((FEAT,), lambda i: (0,)),
            pl.BlockSpec((1, EMB), full),
            pl.BlockSpec((1,), lambda i: (0,)),
        ],
        out_specs=[
            pl.BlockSpec((blk, kn), lambda i: (i, 0)),
            pl.BlockSpec((blk, kn), lambda i: (i, 0)),
            pl.BlockSpec((blk, 128), lambda i: (i, 0)),
            pl.BlockSpec((kn, FEAT), full),
        ],
        out_shape=[
            jax.ShapeDtypeStruct((b, kn), jnp.float32),
            jax.ShapeDtypeStruct((b, kn), jnp.float32),
            jax.ShapeDtypeStruct((b, 128), jnp.float32),
            jax.ShapeDtypeStruct((kn, FEAT), jnp.float32),
        ],
    )(bstu, bexer, k0, k1, k2, wsT, bs, weT, be, wkT, bk, wd, bd)


def _pad_rows(x, npad):
    n = x.shape[0]
    if n == npad:
        return x
    return jnp.pad(x, ((0, npad - n), (0, 0)))


def _prep_edges(idx, val, ep):
    e = idx.shape[1]
    src = idx[1].astype(jnp.int32)
    dst = idx[0].astype(jnp.int32)
    val = val.astype(jnp.float32)
    if e != ep:
        src = jnp.pad(src, (0, ep - e))
        dst = jnp.pad(dst, (0, ep - e))
        val = jnp.pad(val, (0, ep - e))
    shape2 = (ep // 128, 128)
    return jnp.stack([src.reshape(shape2), dst.reshape(shape2)], axis=1), \
        val.reshape(shape2)


def _conv_tables(emb, idx, val, npad, ep, chunk):
    x0 = _pad_rows(emb.astype(jnp.float32), npad)
    edges3, vals2 = _prep_edges(idx, val, ep)
    x1 = _spmm_momentum(edges3, vals2, x0, npad, ep, chunk)
    x2 = _spmm_momentum(edges3, vals2, x1, npad, ep, chunk)
    return x0, x1, x2


def _round_up(n, m):
    return (n + m - 1) // m * m


def kernel(student_id, exercise_id, q_mask, stu_adj_idx, stu_adj_val,
           exer_adj_idx, exer_adj_val, know_adj_idx, know_adj_val,
           stu_emb, exer_emb, know_emb, ki_emb, Ws, bs, We, be, Wk, bk,
           Wd, bd):
    k_num = know_emb.shape[0]

    s_pad = _round_up(stu_emb.shape[0], 512)
    e_pad = _round_up(exer_emb.shape[0], 512)
    k_pad = _round_up(k_num, 512)
    s_chunk, e_chunk, k_chunk = 256, 512, 512
    se_pad = _round_up(stu_adj_idx.shape[1], s_chunk * NS * 2)
    ee_pad = _round_up(exer_adj_idx.shape[1], e_chunk * NS * 2)
    ke_pad = _round_up(know_adj_idx.shape[1], k_chunk * NS * 2)

    s0, s1, s2 = _conv_tables(stu_emb, stu_adj_idx, stu_adj_val, s_pad,
                              se_pad, s_chunk)
    e0, e1, e2 = _conv_tables(exer_emb, exer_adj_idx, exer_adj_val, e_pad,
                              ee_pad, e_chunk)
    k0, k1, k2 = _conv_tables(know_emb, know_adj_idx, know_adj_val, k_pad,
                              ke_pad, k_chunk)

    sid = student_id.astype(jnp.int32)
    eid = exercise_id.astype(jnp.int32)
    bstu, bexer, kib = _batch_lookup(sid, eid, s0, s1, s2, e0, e1, e2,
                                     ki_emb.astype(jnp.float32))

    st, dt, disc2d, kt = _dense_tail(
        bstu, bexer, k0[:k_num], k1[:k_num], k2[:k_num],
        Ws.T, bs, We.T, be, Wk.T, bk, Wd, bd)
    return st, dt, disc2d[:, :1], kt, kib


# speedup vs baseline: 5.7525x; 1.0334x over previous
import functools

import jax
import jax.numpy as jnp
from jax import lax
from jax.experimental import pallas as pl
from jax.experimental.pallas import tpu as pltpu
from jax.experimental.pallas import tpu_sc as plsc

NC = 2
NS = 16
LN = 16
EMB = 32
FEAT = 64
MOM = 0.5
LEAK = 0.1
CHUNK = 256
JR = CHUNK // 128


def _pick_row_chunk(r):
    for c in (128, 112, 64, 32):
        if r % c == 0:
            return c
    return 32


def _spmm_momentum(edges3, vals2, x, npad, ep, chunk):
    n2 = npad // 2
    rpt = n2 // NS
    gt = ep // chunk // NS
    jr = chunk // 128
    ch = _pick_row_chunk(rpt)
    nch = rpt // ch
    grp_n = chunk // LN
    mesh = plsc.VectorSubcoreMesh(core_axis_name="c", subcore_axis_name="s",
                                  num_cores=NC, num_subcores=NS)

    @functools.partial(
        pl.kernel,
        out_type=jax.ShapeDtypeStruct((npad, EMB), jnp.float32),
        mesh=mesh,
        compiler_params=pltpu.CompilerParams(use_tc_tiling_on_sc=False),
        scratch_types=[
            pltpu.VMEM_SHARED((n2 + 16, EMB), jnp.float32),
            pltpu.VMEM((jr, 2, 128), jnp.int32),
            pltpu.VMEM((jr, 2, 128), jnp.int32),
            pltpu.VMEM((jr, 128), jnp.float32),
            pltpu.VMEM((jr, 128), jnp.float32),
            pltpu.VMEM((jr, 128), jnp.int32),
            pltpu.VMEM((jr, 128), jnp.int32),
            pltpu.VMEM((chunk, EMB), jnp.float32),
            pltpu.VMEM((chunk, EMB), jnp.float32),
            pltpu.VMEM((ch, EMB), jnp.float32),
            pltpu.VMEM((ch, EMB), jnp.float32),
            pltpu.SemaphoreType.DMA,
            pltpu.SemaphoreType.DMA,
            pltpu.SemaphoreType.DMA,
            pltpu.SemaphoreType.DMA,
            pltpu.SemaphoreType.DMA,
            pltpu.SemaphoreType.DMA,
        ],
    )
    def k(x_hbm, edges_hbm, vals_hbm, out_hbm,
          acc, idxb0, idxb1, valb0, valb1, lidxb0, lidxb1, rowsb0, rowsb1,
          bufA, bufB, ix_sem0, ix_sem1, g_sem0, g_sem1, s_sem0, s_sem1):
        c = lax.axis_index("c")
        s = lax.axis_index("s")
        base = c * n2
        idxb = (idxb0, idxb1)
        valb = (valb0, valb1)
        lidxb = (lidxb0, lidxb1)
        rowsb = (rowsb0, rowsb1)
        ix_sem = (ix_sem0, ix_sem1)
        g_sem = (g_sem0, g_sem1)
        s_sem = (s_sem0, s_sem1)

        pltpu.sync_copy(x_hbm.at[pl.ds(base + s * rpt, rpt)],
                        acc.at[pl.ds(s * rpt, rpt)])
        plsc.subcore_barrier()

        def issue_idx(cc, b):
            pltpu.async_copy(edges_hbm.at[pl.ds(cc * jr, jr)], idxb[b],
                             ix_sem[b])
            pltpu.async_copy(vals_hbm.at[pl.ds(cc * jr, jr)], valb[b],
                             ix_sem[b])

        def wait_idx(b):
            pltpu.make_async_copy(edges_hbm.at[pl.ds(0, jr)], idxb[b],
                                  ix_sem[b]).wait()
            pltpu.make_async_copy(vals_hbm.at[pl.ds(0, jr)], valb[b],
                                  ix_sem[b]).wait()

        def issue_gather(b):
            for j in range(jr):
                pltpu.async_copy(x_hbm.at[idxb[b].at[j, 0]],
                                 rowsb[b].at[pl.ds(j * 128, 128)], g_sem[b])

        def wait_gather(b):
            for j in range(jr):
                pltpu.make_async_copy(x_hbm.at[idxb[b].at[j, 0]],
                                      rowsb[b].at[pl.ds(j * 128, 128)],
                                      g_sem[b]).wait()

        def issue_scatter(b):
            for j in range(jr):
                pltpu.async_copy(rowsb[b].at[pl.ds(j * 128, 128)],
                                 acc.at[lidxb[b].at[j]], s_sem[b], add=True)

        def wait_scatter(b):
            for j in range(jr):
                pltpu.make_async_copy(rowsb[b].at[pl.ds(j * 128, 128)],
                                      acc.at[lidxb[b].at[j]],
                                      s_sem[b]).wait()

        def scale(b):
            ib = idxb[b]
            vb = valb[b]
            lb = lidxb[b]
            rb = rowsb[b]

            @plsc.parallel_loop(0, grp_n, step=1, unroll=2)
            def grp(kk):
                j = kk // 8
                off = (kk % 8) * LN
                dd = ib[j, 1, pl.ds(off, LN)]
                ll = dd - base
                inr = (ll >= 0) & (ll < n2)
                lb[j, pl.ds(off, LN)] = jnp.where(inr, ll, n2)
                vv = vb[j, pl.ds(off, LN)]
                e0 = kk * LN
                for lane in range(LN):
                    v = vv[lane]
                    e = e0 + lane
                    rb[e, pl.ds(0, LN)] = rb[e, pl.ds(0, LN)] * v
                    rb[e, pl.ds(LN, LN)] = rb[e, pl.ds(LN, LN)] * v

        def chunk_of(i):
            return i * NS + s

        issue_idx(chunk_of(0), 0)
        issue_idx(chunk_of(1), 1)
        wait_idx(0)
        issue_gather(0)

        def body(i2, _):
            i = 2 * i2

            @pl.when(i2 >= 1)
            def _():
                wait_scatter(1)
            wait_idx(1)
            issue_gather(1)

            wait_gather(0)
            scale(0)
            issue_scatter(0)

            @pl.when(i + 2 < gt)
            def _():
                issue_idx(chunk_of(i + 2), 0)
                wait_scatter(0)
                wait_idx(0)
                issue_gather(0)

            wait_gather(1)
            scale(1)
            issue_scatter(1)

            @pl.when(i + 3 < gt)
            def _():
                issue_idx(chunk_of(i + 3), 1)
            return 0

        lax.fori_loop(0, gt // 2, body, 0)
        wait_scatter(0)
        wait_scatter(1)
        plsc.subcore_barrier()

        def out_body(t, _):
            lo = s * rpt + t * ch
            pltpu.sync_copy(acc.at[pl.ds(lo, ch)], bufA)
            pltpu.sync_copy(x_hbm.at[pl.ds(base + lo, ch)], bufB)

            @plsc.parallel_loop(0, ch * 2, step=1, unroll=4)
            def vec(q):
                rr = q // 2
                hh = (q % 2) * LN
                a = bufA[rr, pl.ds(hh, LN)]
                b = bufB[rr, pl.ds(hh, LN)]
                bufA[rr, pl.ds(hh, LN)] = a - (1.0 - MOM) * b
            pltpu.sync_copy(bufA, out_hbm.at[pl.ds(base + lo, ch)])
            return 0

        lax.fori_loop(0, nch, out_body, 0)

    return k(x, edges3, vals2)


def _batch_lookup(sid, eid, s0, s1, s2, e0, e1, e2, ki):
    b = sid.shape[0]
    w = b // (NC * NS)
    mesh = plsc.VectorSubcoreMesh(core_axis_name="c", subcore_axis_name="s",
                                  num_cores=NC, num_subcores=NS)

    @functools.partial(
        pl.kernel,
        out_type=[jax.ShapeDtypeStruct((b, EMB), jnp.float32)] * 3,
        mesh=mesh,
        compiler_params=pltpu.CompilerParams(use_tc_tiling_on_sc=False),
        scratch_types=[
            pltpu.VMEM((w,), jnp.int32),
            pltpu.VMEM((w, EMB), jnp.float32),
            pltpu.VMEM((w, EMB), jnp.float32),
            pltpu.VMEM((w, EMB), jnp.float32),
            pltpu.SemaphoreType.DMA,
        ],
    )
    def k(sid_h, eid_h, s0_h, s1_h, s2_h, e0_h, e1_h, e2_h, ki_h,
          bstu_h, bexer_h, kib_h, idb, g0, g1, g2, sem):
        c = lax.axis_index("c")
        s = lax.axis_index("s")
        wid = s * NC + c
        base = wid * w

        def mean3():
            @plsc.parallel_loop(0, w * 2, step=1, unroll=4)
            def vec(q):
                rr = q // 2
                hh = (q % 2) * LN
                a = g0[rr, pl.ds(hh, LN)]
                bb = g1[rr, pl.ds(hh, LN)]
                cc = g2[rr, pl.ds(hh, LN)]
                g0[rr, pl.ds(hh, LN)] = (a + bb + cc) * (1.0 / 3.0)

        pltpu.sync_copy(sid_h.at[pl.ds(base, w)], idb)
        for tab, dst in ((s0_h, g0), (s1_h, g1), (s2_h, g2)):
            pltpu.async_copy(tab.at[idb], dst, sem).wait()
        mean3()
        pltpu.sync_copy(g0, bstu_h.at[pl.ds(base, w)])

        pltpu.sync_copy(eid_h.at[pl.ds(base, w)], idb)
        for tab, dst in ((e0_h, g0), (e1_h, g1), (e2_h, g2)):
            pltpu.async_copy(tab.at[idb], dst, sem).wait()
        mean3()
        pltpu.sync_copy(g0, bexer_h.at[pl.ds(base, w)])

        pltpu.async_copy(ki_h.at[idb], g1, sem).wait()
        pltpu.sync_copy(g1, kib_h.at[pl.ds(base, w)])

    return k(sid, eid, s0, s1, s2, e0, e1, e2, ki)


def _dense_tail(bstu, bexer, k0, k1, k2, wsT, bs, weT, be, wkT, bk, wd, bd):
    b = bstu.shape[0]
    kn = k0.shape[0]
    blk = 1024
    grid = b // blk

    def body(bs_ref, be_ref, k0_ref, k1_ref, k2_ref, wsT_ref, bsb_ref,
             weT_ref, beb_ref, wkT_ref, bkb_ref, wd_ref, bd_ref,
             st_ref, dt_ref, disc_ref, kt_ref):
        kf = (k0_ref[...] + k1_ref[...] + k2_ref[...]) * (1.0 / 3.0)
        kt = jnp.dot(kf, wkT_ref[...], preferred_element_type=jnp.float32,
                     precision=lax.Precision.HIGHEST) + bkb_ref[...]
        kt = jnp.where(kt > 0, kt, LEAK * kt)
        kt_ref[...] = kt

        st = jnp.dot(bs_ref[...], wsT_ref[...],
                     preferred_element_type=jnp.float32,
                     precision=lax.Precision.HIGHEST) + bsb_ref[...]
        st = jnp.where(st > 0, st, LEAK * st)
        st_ref[...] = lax.dot_general(
            st, kt, (((1,), (1,)), ((), ())),
            preferred_element_type=jnp.float32,
            precision=lax.Precision.HIGHEST)

        dt = jnp.dot(be_ref[...], weT_ref[...],
                     preferred_element_type=jnp.float32,
                     precision=lax.Precision.HIGHEST) + beb_ref[...]
        dt = jnp.where(dt > 0, dt, LEAK * dt)
        dt_ref[...] = lax.dot_general(
            dt, kt, (((1,), (1,)), ((), ())),
            preferred_element_type=jnp.float32,
            precision=lax.Precision.HIGHEST)

        dv = jnp.sum(be_ref[...] * wd_ref[...], axis=1) + bd_ref[0]
        disc_ref[...] = jax.nn.sigmoid(dv)[:, None] * jnp.ones(
            (1, 128), jnp.float32)

    full = lambda i: (0, 0)
    return pl.pallas_call(
        body,
        grid=(grid,),
        in_specs=[
            pl.BlockSpec((blk, EMB), lambda i: (i, 0)),
            pl.BlockSpec((blk, EMB), lambda i: (i, 0)),
            pl.BlockSpec((kn, EMB), full),
            pl.BlockSpec((kn, EMB), full),
            pl.BlockSpec((kn, EMB), full),
            pl.BlockSpec((EMB, FEAT), full),
            pl.BlockSpec((FEAT,), lambda i: (0,)),
            pl.BlockSpec((EMB, FEAT), full),
            pl.BlockSpec((FEAT,), lambda i: (0,)),
            pl.BlockSpec((EMB, FEAT), full),
            pl.BlockSpec((FEAT,), lambda i: (0,)),
            pl.BlockSpec((1, EMB), full),
            pl.BlockSpec((1,), lambda i: (0,)),
        ],
        out_specs=[
            pl.BlockSpec((blk, kn), lambda i: (i, 0)),
            pl.BlockSpec((blk, kn), lambda i: (i, 0)),
            pl.BlockSpec((blk, 128), lambda i: (i, 0)),
            pl.BlockSpec((kn, FEAT), full),
        ],
        out_shape=[
            jax.ShapeDtypeStruct((b, kn), jnp.float32),
            jax.ShapeDtypeStruct((b, kn), jnp.float32),
            jax.ShapeDtypeStruct((b, 128), jnp.float32),
            jax.ShapeDtypeStruct((kn, FEAT), jnp.float32),
        ],
    )(bstu, bexer, k0, k1, k2, wsT, bs, weT, be, wkT, bk, wd, bd)


def _pad_rows(x, npad):
    n = x.shape[0]
    if n == npad:
        return x
    return jnp.pad(x, ((0, npad - n), (0, 0)))


def _prep_edges(idx, val, ep):
    e = idx.shape[1]
    src = idx[1].astype(jnp.int32)
    dst = idx[0].astype(jnp.int32)
    val = val.astype(jnp.float32)
    if e != ep:
        src = jnp.pad(src, (0, ep - e))
        dst = jnp.pad(dst, (0, ep - e))
        val = jnp.pad(val, (0, ep - e))
    shape2 = (ep // 128, 128)
    return jnp.stack([src.reshape(shape2), dst.reshape(shape2)], axis=1), \
        val.reshape(shape2)


def _conv_tables(emb, idx, val, npad, ep, chunk):
    x0 = _pad_rows(emb.astype(jnp.float32), npad)
    edges3, vals2 = _prep_edges(idx, val, ep)
    x1 = _spmm_momentum(edges3, vals2, x0, npad, ep, chunk)
    x2 = _spmm_momentum(edges3, vals2, x1, npad, ep, chunk)
    return x0, x1, x2


def _round_up(n, m):
    return (n + m - 1) // m * m


def kernel(student_id, exercise_id, q_mask, stu_adj_idx, stu_adj_val,
           exer_adj_idx, exer_adj_val, know_adj_idx, know_adj_val,
           stu_emb, exer_emb, know_emb, ki_emb, Ws, bs, We, be, Wk, bk,
           Wd, bd):
    k_num = know_emb.shape[0]

    s_pad = _round_up(stu_emb.shape[0], 512)
    e_pad = _round_up(exer_emb.shape[0], 512)
    k_pad = _round_up(k_num, 512)
    s_chunk, e_chunk, k_chunk = 256, 512, 512
    se_pad = _round_up(stu_adj_idx.shape[1], s_chunk * NS * 2)
    ee_pad = _round_up(exer_adj_idx.shape[1], e_chunk * NS * 2)
    ke_pad = _round_up(know_adj_idx.shape[1], k_chunk * NS * 2)

    s0, s1, s2 = _conv_tables(stu_emb, stu_adj_idx, stu_adj_val, s_pad,
                              se_pad, s_chunk)
    e0, e1, e2 = _conv_tables(exer_emb, exer_adj_idx, exer_adj_val, e_pad,
                              ee_pad, e_chunk)
    k0, k1, k2 = _conv_tables(know_emb, know_adj_idx, know_adj_val, k_pad,
                              ke_pad, k_chunk)

    sid = student_id.astype(jnp.int32)
    eid = exercise_id.astype(jnp.int32)
    bstu, bexer, kib = _batch_lookup(sid, eid, s0, s1, s2, e0, e1, e2,
                                     ki_emb.astype(jnp.float32))

    st, dt, disc2d, kt = _dense_tail(
        bstu, bexer, k0[:k_num], k1[:k_num], k2[:k_num],
        Ws.T, bs, We.T, be, Wk.T, bk, Wd, bd)
    return st, dt, disc2d[:, :1], kt, kib


# speedup vs baseline: 11.7945x; 2.0503x over previous
import functools

import jax
import jax.numpy as jnp
from jax import lax
from jax.experimental import pallas as pl
from jax.experimental.pallas import tpu as pltpu
from jax.experimental.pallas import tpu_sc as plsc

NC = 2
NS = 16
LN = 16
EMB = 32
FEAT = 64
MOM = 0.5
LEAK = 0.1
CHUNK = 256
JR = CHUNK // 128


def _pick_row_chunk(r):
    for c in (128, 112, 64, 32):
        if r % c == 0:
            return c
    return 32


def _spmm_momentum(edges3, vals2, x, npad, ep, chunk):
    n2 = npad // 2
    rpt = n2 // NS
    gt = ep // chunk // NS
    jr = chunk // 128
    ch = _pick_row_chunk(rpt)
    nch = rpt // ch
    grp_n = chunk // LN
    mesh = plsc.VectorSubcoreMesh(core_axis_name="c", subcore_axis_name="s",
                                  num_cores=NC, num_subcores=NS)

    @functools.partial(
        pl.kernel,
        out_type=jax.ShapeDtypeStruct((npad, EMB), jnp.float32),
        mesh=mesh,
        compiler_params=pltpu.CompilerParams(use_tc_tiling_on_sc=False),
        scratch_types=[
            pltpu.VMEM_SHARED((n2, EMB), jnp.float32),
            pltpu.VMEM((jr, 2, 128), jnp.int32),
            pltpu.VMEM((jr, 2, 128), jnp.int32),
            pltpu.VMEM((jr, 128), jnp.float32),
            pltpu.VMEM((jr, 128), jnp.float32),
            pltpu.VMEM((jr, 128), jnp.int32),
            pltpu.VMEM((jr, 128), jnp.int32),
            pltpu.VMEM((chunk, EMB), jnp.float32),
            pltpu.VMEM((chunk, EMB), jnp.float32),
            pltpu.VMEM((ch, EMB), jnp.float32),
            pltpu.VMEM((ch, EMB), jnp.float32),
            pltpu.SemaphoreType.DMA,
            pltpu.SemaphoreType.DMA,
            pltpu.SemaphoreType.DMA,
            pltpu.SemaphoreType.DMA,
            pltpu.SemaphoreType.DMA,
            pltpu.SemaphoreType.DMA,
        ],
    )
    def k(x_hbm, edges_hbm, vals_hbm, out_hbm,
          acc, idxb0, idxb1, valb0, valb1, lidxb0, lidxb1, rowsb0, rowsb1,
          bufA, bufB, ix_sem0, ix_sem1, g_sem0, g_sem1, s_sem0, s_sem1):
        c = lax.axis_index("c")
        s = lax.axis_index("s")
        base = c * n2
        idxb = (idxb0, idxb1)
        valb = (valb0, valb1)
        lidxb = (lidxb0, lidxb1)
        rowsb = (rowsb0, rowsb1)
        ix_sem = (ix_sem0, ix_sem1)
        g_sem = (g_sem0, g_sem1)
        s_sem = (s_sem0, s_sem1)

        pltpu.sync_copy(x_hbm.at[pl.ds(base + s * rpt, rpt)],
                        acc.at[pl.ds(s * rpt, rpt)])
        plsc.subcore_barrier()

        def issue_idx(cc, b):
            pltpu.async_copy(edges_hbm.at[pl.ds(cc * jr, jr)], idxb[b],
                             ix_sem[b])
            pltpu.async_copy(vals_hbm.at[pl.ds(cc * jr, jr)], valb[b],
                             ix_sem[b])

        def wait_idx(b):
            pltpu.make_async_copy(edges_hbm.at[pl.ds(0, jr)], idxb[b],
                                  ix_sem[b]).wait()
            pltpu.make_async_copy(vals_hbm.at[pl.ds(0, jr)], valb[b],
                                  ix_sem[b]).wait()

        def issue_gather(b):
            for j in range(jr):
                pltpu.async_copy(x_hbm.at[idxb[b].at[j, 0]],
                                 rowsb[b].at[pl.ds(j * 128, 128)], g_sem[b])

        def wait_gather(b):
            for j in range(jr):
                pltpu.make_async_copy(x_hbm.at[idxb[b].at[j, 0]],
                                      rowsb[b].at[pl.ds(j * 128, 128)],
                                      g_sem[b]).wait()

        def issue_scatter(b):
            for j in range(jr):
                pltpu.async_copy(rowsb[b].at[pl.ds(j * 128, 128)],
                                 acc.at[lidxb[b].at[j]], s_sem[b], add=True)

        def wait_scatter(b):
            for j in range(jr):
                pltpu.make_async_copy(rowsb[b].at[pl.ds(j * 128, 128)],
                                      acc.at[lidxb[b].at[j]],
                                      s_sem[b]).wait()

        def scale(b):
            ib = idxb[b]
            vb = valb[b]
            lb = lidxb[b]
            rb = rowsb[b]

            @plsc.parallel_loop(0, grp_n, step=1, unroll=2)
            def grp(kk):
                j = kk // 8
                off = (kk % 8) * LN
                dd = ib[j, 1, pl.ds(off, LN)]
                lb[j, pl.ds(off, LN)] = dd - jnp.where(dd >= n2, n2, 0)
                inr = (dd >= base) & (dd < base + n2)
                vv = jnp.where(inr, vb[j, pl.ds(off, LN)], 0.0)
                e0 = kk * LN
                for lane in range(LN):
                    v = vv[lane]
                    e = e0 + lane
                    rb[e, pl.ds(0, LN)] = rb[e, pl.ds(0, LN)] * v
                    rb[e, pl.ds(LN, LN)] = rb[e, pl.ds(LN, LN)] * v

        def chunk_of(i):
            return i * NS + s

        issue_idx(chunk_of(0), 0)
        issue_idx(chunk_of(1), 1)
        wait_idx(0)
        issue_gather(0)

        def body(i2, _):
            i = 2 * i2

            @pl.when(i2 >= 1)
            def _():
                wait_scatter(1)
            wait_idx(1)
            issue_gather(1)

            wait_gather(0)
            scale(0)
            issue_scatter(0)

            @pl.when(i + 2 < gt)
            def _():
                issue_idx(chunk_of(i + 2), 0)
                wait_scatter(0)
                wait_idx(0)
                issue_gather(0)

            wait_gather(1)
            scale(1)
            issue_scatter(1)

            @pl.when(i + 3 < gt)
            def _():
                issue_idx(chunk_of(i + 3), 1)
            return 0

        lax.fori_loop(0, gt // 2, body, 0)
        wait_scatter(0)
        wait_scatter(1)
        plsc.subcore_barrier()

        def out_body(t, _):
            lo = s * rpt + t * ch
            pltpu.sync_copy(acc.at[pl.ds(lo, ch)], bufA)
            pltpu.sync_copy(x_hbm.at[pl.ds(base + lo, ch)], bufB)

            @plsc.parallel_loop(0, ch * 2, step=1, unroll=4)
            def vec(q):
                rr = q // 2
                hh = (q % 2) * LN
                a = bufA[rr, pl.ds(hh, LN)]
                b = bufB[rr, pl.ds(hh, LN)]
                bufA[rr, pl.ds(hh, LN)] = a - (1.0 - MOM) * b
            pltpu.sync_copy(bufA, out_hbm.at[pl.ds(base + lo, ch)])
            return 0

        lax.fori_loop(0, nch, out_body, 0)

    return k(x, edges3, vals2)


def _batch_lookup(sid, eid, s0, s1, s2, e0, e1, e2, ki):
    b = sid.shape[0]
    w = b // (NC * NS)
    mesh = plsc.VectorSubcoreMesh(core_axis_name="c", subcore_axis_name="s",
                                  num_cores=NC, num_subcores=NS)

    @functools.partial(
        pl.kernel,
        out_type=[jax.ShapeDtypeStruct((b, EMB), jnp.float32)] * 3,
        mesh=mesh,
        compiler_params=pltpu.CompilerParams(use_tc_tiling_on_sc=False),
        scratch_types=[
            pltpu.VMEM((w,), jnp.int32),
            pltpu.VMEM((w, EMB), jnp.float32),
            pltpu.VMEM((w, EMB), jnp.float32),
            pltpu.VMEM((w, EMB), jnp.float32),
            pltpu.SemaphoreType.DMA,
        ],
    )
    def k(sid_h, eid_h, s0_h, s1_h, s2_h, e0_h, e1_h, e2_h, ki_h,
          bstu_h, bexer_h, kib_h, idb, g0, g1, g2, sem):
        c = lax.axis_index("c")
        s = lax.axis_index("s")
        wid = s * NC + c
        base = wid * w

        def mean3():
            @plsc.parallel_loop(0, w * 2, step=1, unroll=4)
            def vec(q):
                rr = q // 2
                hh = (q % 2) * LN
                a = g0[rr, pl.ds(hh, LN)]
                bb = g1[rr, pl.ds(hh, LN)]
                cc = g2[rr, pl.ds(hh, LN)]
                g0[rr, pl.ds(hh, LN)] = (a + bb + cc) * (1.0 / 3.0)

        pltpu.sync_copy(sid_h.at[pl.ds(base, w)], idb)
        for tab, dst in ((s0_h, g0), (s1_h, g1), (s2_h, g2)):
            pltpu.async_copy(tab.at[idb], dst, sem).wait()
        mean3()
        pltpu.sync_copy(g0, bstu_h.at[pl.ds(base, w)])

        pltpu.sync_copy(eid_h.at[pl.ds(base, w)], idb)
        for tab, dst in ((e0_h, g0), (e1_h, g1), (e2_h, g2)):
            pltpu.async_copy(tab.at[idb], dst, sem).wait()
        mean3()
        pltpu.sync_copy(g0, bexer_h.at[pl.ds(base, w)])

        pltpu.async_copy(ki_h.at[idb], g1, sem).wait()
        pltpu.sync_copy(g1, kib_h.at[pl.ds(base, w)])

    return k(sid, eid, s0, s1, s2, e0, e1, e2, ki)


def _dense_tail(bstu, bexer, k0, k1, k2, wsT, bs, weT, be, wkT, bk, wd, bd):
    b = bstu.shape[0]
    kn = k0.shape[0]
    blk = 1024
    grid = b // blk

    def body(bs_ref, be_ref, k0_ref, k1_ref, k2_ref, wsT_ref, bsb_ref,
             weT_ref, beb_ref, wkT_ref, bkb_ref, wd_ref, bd_ref,
             st_ref, dt_ref, disc_ref, kt_ref):
        kf = (k0_ref[...] + k1_ref[...] + k2_ref[...]) * (1.0 / 3.0)
        kt = jnp.dot(kf, wkT_ref[...], preferred_element_type=jnp.float32,
                     precision=lax.Precision.HIGHEST) + bkb_ref[...]
        kt = jnp.where(kt > 0, kt, LEAK * kt)
        kt_ref[...] = kt

        st = jnp.dot(bs_ref[...], wsT_ref[...],
                     preferred_element_type=jnp.float32,
                     precision=lax.Precision.HIGHEST) + bsb_ref[...]
        st = jnp.where(st > 0, st, LEAK * st)
        st_ref[...] = lax.dot_general(
            st, kt, (((1,), (1,)), ((), ())),
            preferred_element_type=jnp.float32,
            precision=lax.Precision.HIGHEST)

        dt = jnp.dot(be_ref[...], weT_ref[...],
                     preferred_element_type=jnp.float32,
                     precision=lax.Precision.HIGHEST) + beb_ref[...]
        dt = jnp.where(dt > 0, dt, LEAK * dt)
        dt_ref[...] = lax.dot_general(
            dt, kt, (((1,), (1,)), ((), ())),
            preferred_element_type=jnp.float32,
            precision=lax.Precision.HIGHEST)

        dv = jnp.sum(be_ref[...] * wd_ref[...], axis=1) + bd_ref[0]
        disc_ref[...] = jax.nn.sigmoid(dv)[:, None] * jnp.ones(
            (1, 128), jnp.float32)

    full = lambda i: (0, 0)
    return pl.pallas_call(
        body,
        grid=(grid,),
        in_specs=[
            pl.BlockSpec((blk, EMB), lambda i: (i, 0)),
            pl.BlockSpec((blk, EMB), lambda i: (i, 0)),
            pl.BlockSpec((kn, EMB), full),
            pl.BlockSpec((kn, EMB), full),
            pl.BlockSpec((kn, EMB), full),
            pl.BlockSpec((EMB, FEAT), full),
            pl.BlockSpec((FEAT,), lambda i: (0,)),
            pl.BlockSpec((EMB, FEAT), full),
            pl.BlockSpec((FEAT,), lambda i: (0,)),
            pl.BlockSpec((EMB, FEAT), full),
            pl.BlockSpec((FEAT,), lambda i: (0,)),
            pl.BlockSpec((1, EMB), full),
            pl.BlockSpec((1,), lambda i: (0,)),
        ],
        out_specs=[
            pl.BlockSpec((blk, kn), lambda i: (i, 0)),
            pl.BlockSpec((blk, kn), lambda i: (i, 0)),
            pl.BlockSpec((blk, 128), lambda i: (i, 0)),
            pl.BlockSpec((kn, FEAT), full),
        ],
        out_shape=[
            jax.ShapeDtypeStruct((b, kn), jnp.float32),
            jax.ShapeDtypeStruct((b, kn), jnp.float32),
            jax.ShapeDtypeStruct((b, 128), jnp.float32),
            jax.ShapeDtypeStruct((kn, FEAT), jnp.float32),
        ],
    )(bstu, bexer, k0, k1, k2, wsT, bs, weT, be, wkT, bk, wd, bd)


def _pad_rows(x, npad):
    n = x.shape[0]
    if n == npad:
        return x
    return jnp.pad(x, ((0, npad - n), (0, 0)))


def _prep_edges(idx, val, ep):
    e = idx.shape[1]
    src = idx[1].astype(jnp.int32)
    dst = idx[0].astype(jnp.int32)
    val = val.astype(jnp.float32)
    if e != ep:
        src = jnp.pad(src, (0, ep - e))
        dst = jnp.pad(dst, (0, ep - e))
        val = jnp.pad(val, (0, ep - e))
    shape2 = (ep // 128, 128)
    return jnp.stack([src.reshape(shape2), dst.reshape(shape2)], axis=1), \
        val.reshape(shape2)


def _conv_tables(emb, idx, val, npad, ep, chunk):
    x0 = _pad_rows(emb.astype(jnp.float32), npad)
    edges3, vals2 = _prep_edges(idx, val, ep)
    x1 = _spmm_momentum(edges3, vals2, x0, npad, ep, chunk)
    x2 = _spmm_momentum(edges3, vals2, x1, npad, ep, chunk)
    return x0, x1, x2


def _round_up(n, m):
    return (n + m - 1) // m * m


def kernel(student_id, exercise_id, q_mask, stu_adj_idx, stu_adj_val,
           exer_adj_idx, exer_adj_val, know_adj_idx, know_adj_val,
           stu_emb, exer_emb, know_emb, ki_emb, Ws, bs, We, be, Wk, bk,
           Wd, bd):
    k_num = know_emb.shape[0]

    s_pad = _round_up(stu_emb.shape[0], 512)
    e_pad = _round_up(exer_emb.shape[0], 512)
    k_pad = _round_up(k_num, 512)
    s_chunk, e_chunk, k_chunk = 256, 512, 512
    se_pad = _round_up(stu_adj_idx.shape[1], s_chunk * NS * 2)
    ee_pad = _round_up(exer_adj_idx.shape[1], e_chunk * NS * 2)
    ke_pad = _round_up(know_adj_idx.shape[1], k_chunk * NS * 2)

    s0, s1, s2 = _conv_tables(stu_emb, stu_adj_idx, stu_adj_val, s_pad,
                              se_pad, s_chunk)
    e0, e1, e2 = _conv_tables(exer_emb, exer_adj_idx, exer_adj_val, e_pad,
                              ee_pad, e_chunk)
    k0, k1, k2 = _conv_tables(know_emb, know_adj_idx, know_adj_val, k_pad,
                              ke_pad, k_chunk)

    sid = student_id.astype(jnp.int32)
    eid = exercise_id.astype(jnp.int32)
    bstu, bexer, kib = _batch_lookup(sid, eid, s0, s1, s2, e0, e1, e2,
                                     ki_emb.astype(jnp.float32))

    st, dt, disc2d, kt = _dense_tail(
        bstu, bexer, k0[:k_num], k1[:k_num], k2[:k_num],
        Ws.T, bs, We.T, be, Wk.T, bk, Wd, bd)
    return st, dt, disc2d[:, :1], kt, kib
